# baseline (speedup 1.0000x reference)
import functools

import jax
import jax.numpy as jnp
from jax import lax
from jax.experimental import pallas as pl
from jax.experimental.pallas import tpu as pltpu
from jax.experimental.pallas import tpu_sc as plsc

N = 10000
E = 320000
NFEAT = 128
NHID = 64
NCLASS = 40
HEADS = 8

C2P = 48
W1ROW = 80
W2ROW = 64

BN = 400
CH = 80
NTILES = 16
EPT = E // NTILES
EPT2 = E // (2 * NTILES)
RPT = N // NTILES
ZR = 25

@functools.lru_cache(maxsize=None)
def _mesh():
    return plsc.VectorSubcoreMesh(
        core_axis_name="c", subcore_axis_name="s", num_cores=2, num_subcores=16)


def _k1_body(x_ref, w1_ref, as_ref, ad_ref, h1_ref, asrc_ref, adst_ref):
    xb = x_ref[...]
    wh = w1_ref[0]
    hh = jnp.dot(xb, wh, preferred_element_type=jnp.float32)
    h1_ref[0] = hh
    asrc_ref[...] = jnp.sum(hh * as_ref[0], axis=1)[None, None, :]
    adst_ref[...] = jnp.sum(hh * ad_ref[0], axis=1)[None, None, :]


def _k1(x, W1, a_src1, a_dst1):
    return pl.pallas_call(
        _k1_body,
        grid=(HEADS,),
        in_specs=[
            pl.BlockSpec((N, NFEAT), lambda h: (0, 0)),
            pl.BlockSpec((1, NFEAT, NHID), lambda h: (h, 0, 0)),
            pl.BlockSpec((1, 1, NHID), lambda h: (h, 0, 0)),
            pl.BlockSpec((1, 1, NHID), lambda h: (h, 0, 0)),
        ],
        out_specs=[
            pl.BlockSpec((1, N, NHID), lambda h: (h, 0, 0)),
            pl.BlockSpec((1, 1, N), lambda h: (h, 0, 0)),
            pl.BlockSpec((1, 1, N), lambda h: (h, 0, 0)),
        ],
        out_shape=[
            jax.ShapeDtypeStruct((HEADS, N, NHID), jnp.float32),
            jax.ShapeDtypeStruct((HEADS, 1, N), jnp.float32),
            jax.ShapeDtypeStruct((HEADS, 1, N), jnp.float32),
        ],
    )(x, W1.transpose(1, 0, 2), a_src1.reshape(HEADS, 1, NHID),
      a_dst1.reshape(HEADS, 1, NHID))


def _zero2d(ref, nrows, ncols):
    z16 = jnp.zeros((16,), jnp.float32)

    def body(r, _):
        for cc in range(ncols // 16):
            ref[r, pl.ds(cc * 16, 16)] = z16
        return _

    lax.fori_loop(0, nrows, body, None)


def _s1_body(asrc_hbm, adst_hbm, src_hbm, dst_hbm, h1t_hbm, out_hbm,
             asrc_v, adst_v, srcb0, srcb1, srcb2, dstb0, dstb1, dstb2,
             idxb0, idxb1, idxb2, gb0, gb1, gb2, stb0, stb1, stb2,
             ee_v, zbuf_v, num_sh, gs0, gs1, gs2, ss0, ss1, ss2):
    c = lax.axis_index("c")
    s = lax.axis_index("s")
    srcb = (srcb0, srcb1, srcb2)
    dstb = (dstb0, dstb1, dstb2)
    idxb = (idxb0, idxb1, idxb2)
    gb = (gb0, gb1, gb2)
    stb = (stb0, stb1, stb2)
    gsem = (gs0, gs1, gs2)
    ssem = (ss0, ss1, ss2)
    NCH = EPT // CH
    NG = CH // 16

    _zero2d(zbuf_v, ZR, W1ROW)
    for b in range(3):
        _zero2d(stb[b], CH, W1ROW)

    def load_chunk(b, k, hbase):
        ebase = s * EPT + k * CH
        pltpu.sync_copy(src_hbm.at[pl.ds(ebase, CH)], srcb[b])
        pltpu.sync_copy(dst_hbm.at[pl.ds(ebase, CH)], dstb[b])

        def bidx(gi, _i):
            idxb[b][pl.ds(gi * 16, 16)] = srcb[b][pl.ds(gi * 16, 16)] + hbase
            return _i

        lax.fori_loop(0, NG, bidx, None)
        pltpu.async_copy(h1t_hbm.at[idxb[b]], gb[b], gsem[b])

    def wait_gather(b):
        pltpu.make_async_copy(h1t_hbm.at[idxb[b]], gb[b], gsem[b]).wait()

    def issue_scatter(b):
        pltpu.async_copy(stb[b], num_sh.at[dstb[b]], ssem[b], add=True)

    def wait_scatter(b):
        pltpu.make_async_copy(stb[b], num_sh.at[dstb[b]], ssem[b]).wait()

    def compute(b):
        def eeg(gi, _i):
            sv = srcb[b][pl.ds(gi * 16, 16)]
            dv = dstb[b][pl.ds(gi * 16, 16)]
            z = plsc.load_gather(asrc_v, [sv]) + plsc.load_gather(adst_v, [dv])
            ee_v[pl.ds(gi * 16, 16)] = jnp.exp(jnp.maximum(z, z * 0.2))
            return _i

        lax.fori_loop(0, NG, eeg, None)

        iota16 = lax.iota(jnp.int32, 16)
        dencol = jnp.full((16,), NHID, jnp.int32)

        @plsc.parallel_loop(0, NG, 1)
        def pe(gi):
            r0 = gi * 16
            ee16 = ee_v[pl.ds(r0, 16)]
            plsc.store_scatter(stb[b], [iota16 + r0, dencol], ee16)
            for j in range(16):
                e = r0 + j
                eev = plsc.load_gather(ee_v, [jnp.full((16,), e, jnp.int32)])
                for cc in range(NHID // 16):
                    stb[b][e, pl.ds(cc * 16, 16)] = (
                        gb[b][e, pl.ds(cc * 16, 16)] * eev)

    def head_body(hp, _):
        h = c * (HEADS // 2) + hp
        hbase = h * N
        def zz_body(zz, _z):
            pltpu.sync_copy(zbuf_v, num_sh.at[pl.ds(s * RPT + zz * ZR, ZR)])
            return _z

        lax.fori_loop(0, RPT // ZR, zz_body, None)
        pltpu.sync_copy(asrc_hbm.at[h, 0], asrc_v)
        pltpu.sync_copy(adst_hbm.at[h, 0], adst_v)
        plsc.subcore_barrier()

        load_chunk(0, 0, hbase)

        def triple(p, _t):
            for b in range(3):
                k = 3 * p + b

                @pl.when(jnp.logical_and(k >= 2, k <= NCH + 1))
                def _w():
                    wait_scatter((b + 1) % 3)

                @pl.when(k + 1 < NCH)
                def _l():
                    load_chunk((b + 1) % 3, k + 1, hbase)

                @pl.when(k < NCH)
                def _c():
                    wait_gather(b)
                    compute(b)
                    issue_scatter(b)
            return _t

        lax.fori_loop(0, (NCH + 4) // 3, triple, None)
        plsc.subcore_barrier()
        pltpu.sync_copy(num_sh.at[pl.ds(s * RPT, RPT)], out_hbm.at[h, s])
        plsc.subcore_barrier()
        return _

    lax.fori_loop(0, HEADS // 2, head_body, None)


def _s1(asrc1, adst1, src, dst, h1t_flat):
    f = pl.kernel(
        _s1_body,
        out_type=jax.ShapeDtypeStruct((HEADS, NTILES, RPT, W1ROW), jnp.float32),
        mesh=_mesh(),
        compiler_params=pltpu.CompilerParams(
            use_tc_tiling_on_sc=False, needs_layout_passes=False),
        scratch_types=[
            pltpu.VMEM((N,), jnp.float32),
            pltpu.VMEM((N,), jnp.float32),
            pltpu.VMEM((CH,), jnp.int32),
            pltpu.VMEM((CH,), jnp.int32),
            pltpu.VMEM((CH,), jnp.int32),
            pltpu.VMEM((CH,), jnp.int32),
            pltpu.VMEM((CH,), jnp.int32),
            pltpu.VMEM((CH,), jnp.int32),
            pltpu.VMEM((CH,), jnp.int32),
            pltpu.VMEM((CH,), jnp.int32),
            pltpu.VMEM((CH,), jnp.int32),
            pltpu.VMEM((CH, NHID), jnp.float32),
            pltpu.VMEM((CH, NHID), jnp.float32),
            pltpu.VMEM((CH, NHID), jnp.float32),
            pltpu.VMEM((CH, W1ROW), jnp.float32),
            pltpu.VMEM((CH, W1ROW), jnp.float32),
            pltpu.VMEM((CH, W1ROW), jnp.float32),
            pltpu.VMEM((CH,), jnp.float32),
            pltpu.VMEM((ZR, W1ROW), jnp.float32),
            pltpu.VMEM_SHARED((N, W1ROW), jnp.float32),
            pltpu.SemaphoreType.DMA,
            pltpu.SemaphoreType.DMA,
            pltpu.SemaphoreType.DMA,
            pltpu.SemaphoreType.DMA,
            pltpu.SemaphoreType.DMA,
            pltpu.SemaphoreType.DMA,
        ],
    )
    return f(asrc1, adst1, src, dst, h1t_flat)


def _k2a_body(num_ref, b1_ref, w2_ref, h2_ref):
    acc = num_ref[...]
    den = acc[:, :, NHID : NHID + 1]
    o = jnp.sum(acc[:, :, :NHID] / (den + 1e-16), axis=0) * (1.0 / HEADS)
    o = o + b1_ref[...]
    h2a = jnp.where(o > 0, o, jnp.exp(o) - 1.0)
    h2_ref[...] = jnp.dot(h2a, w2_ref[...], preferred_element_type=jnp.float32)


def _k2a(num1, b1, W2p):
    nb = N // BN
    return pl.pallas_call(
        _k2a_body,
        grid=(nb,),
        in_specs=[
            pl.BlockSpec((HEADS, BN, W1ROW), lambda i: (0, i, 0)),
            pl.BlockSpec((1, NHID), lambda i: (0, 0)),
            pl.BlockSpec((NHID, C2P), lambda i: (0, 0)),
        ],
        out_specs=pl.BlockSpec((BN, C2P), lambda i: (i, 0)),
        out_shape=jax.ShapeDtypeStruct((N, C2P), jnp.float32),
    )(num1, b1, W2p)


def _k2b_body(h2_ref, a2s_ref, a2d_ref, asrc2_ref, adst2_ref):
    h2 = h2_ref[...]
    asrc2_ref[...] = jnp.sum(h2 * a2s_ref[0][None, :], axis=1)[None, None, :]
    adst2_ref[...] = jnp.sum(h2 * a2d_ref[0][None, :], axis=1)[None, None, :]


def _k2b(h2t, a2sp, a2dp):
    return pl.pallas_call(
        _k2b_body,
        grid=(1,),
        in_specs=[
            pl.BlockSpec((N, C2P), lambda i: (0, 0)),
            pl.BlockSpec((1, C2P), lambda i: (0, 0)),
            pl.BlockSpec((1, C2P), lambda i: (0, 0)),
        ],
        out_specs=[
            pl.BlockSpec((1, 1, N), lambda i: (0, 0, 0)),
            pl.BlockSpec((1, 1, N), lambda i: (0, 0, 0)),
        ],
        out_shape=[
            jax.ShapeDtypeStruct((1, 1, N), jnp.float32),
            jax.ShapeDtypeStruct((1, 1, N), jnp.float32),
        ],
    )(h2t, a2sp, a2dp)


def _s2_body(asrc_hbm, adst_hbm, src_hbm, dst_hbm, h2t_hbm, out_hbm,
             asrc_v, adst_v, srcb0, srcb1, srcb2, dstb0, dstb1, dstb2,
             gb0, gb1, gb2, stb0, stb1, stb2,
             ee_v, zbuf_v, num_sh, gs0, gs1, gs2, ss0, ss1, ss2):
    c = lax.axis_index("c")
    s = lax.axis_index("s")
    srcb = (srcb0, srcb1, srcb2)
    dstb = (dstb0, dstb1, dstb2)
    gb = (gb0, gb1, gb2)
    stb = (stb0, stb1, stb2)
    gsem = (gs0, gs1, gs2)
    ssem = (ss0, ss1, ss2)
    NCH = EPT2 // CH
    NG = CH // 16

    _zero2d(zbuf_v, ZR, W2ROW)
    for b in range(3):
        _zero2d(stb[b], CH, W2ROW)

    def zz_body(zz, _z):
        pltpu.sync_copy(zbuf_v, num_sh.at[pl.ds(s * RPT + zz * ZR, ZR)])
        return _z

    lax.fori_loop(0, RPT // ZR, zz_body, None)
    pltpu.sync_copy(asrc_hbm.at[0, 0], asrc_v)
    pltpu.sync_copy(adst_hbm.at[0, 0], adst_v)
    plsc.subcore_barrier()

    def load_chunk(b, k):
        ebase = (c * NTILES + s) * EPT2 + k * CH
        pltpu.sync_copy(src_hbm.at[pl.ds(ebase, CH)], srcb[b])
        pltpu.sync_copy(dst_hbm.at[pl.ds(ebase, CH)], dstb[b])
        pltpu.async_copy(h2t_hbm.at[srcb[b]], gb[b], gsem[b])

    def wait_gather(b):
        pltpu.make_async_copy(h2t_hbm.at[srcb[b]], gb[b], gsem[b]).wait()

    def issue_scatter(b):
        pltpu.async_copy(stb[b], num_sh.at[dstb[b]], ssem[b], add=True)

    def wait_scatter(b):
        pltpu.make_async_copy(stb[b], num_sh.at[dstb[b]], ssem[b]).wait()

    def compute(b):
        def eeg(gi, _i):
            sv = srcb[b][pl.ds(gi * 16, 16)]
            dv = dstb[b][pl.ds(gi * 16, 16)]
            z = plsc.load_gather(asrc_v, [sv]) + plsc.load_gather(adst_v, [dv])
            ee_v[pl.ds(gi * 16, 16)] = jnp.exp(jnp.maximum(z, z * 0.2))
            return _i

        lax.fori_loop(0, NG, eeg, None)

        iota16 = lax.iota(jnp.int32, 16)
        dencol = jnp.full((16,), C2P, jnp.int32)

        @plsc.parallel_loop(0, NG, 1)
        def pe(gi):
            r0 = gi * 16
            ee16 = ee_v[pl.ds(r0, 16)]
            plsc.store_scatter(stb[b], [iota16 + r0, dencol], ee16)
            for j in range(16):
                e = r0 + j
                eev = plsc.load_gather(ee_v, [jnp.full((16,), e, jnp.int32)])
                for cc in range(C2P // 16):
                    stb[b][e, pl.ds(cc * 16, 16)] = (
                        gb[b][e, pl.ds(cc * 16, 16)] * eev)

    load_chunk(0, 0)

    def triple(p, _t):
        for b in range(3):
            k = 3 * p + b

            @pl.when(jnp.logical_and(k >= 2, k <= NCH + 1))
            def _w():
                wait_scatter((b + 1) % 3)

            @pl.when(k + 1 < NCH)
            def _l():
                load_chunk((b + 1) % 3, k + 1)

            @pl.when(k < NCH)
            def _c():
                wait_gather(b)
                compute(b)
                issue_scatter(b)
        return _t

    lax.fori_loop(0, (NCH + 4) // 3, triple, None)
    plsc.subcore_barrier()
    pltpu.sync_copy(num_sh.at[pl.ds(s * RPT, RPT)], out_hbm.at[c, s])


def _s2(asrc2, adst2, src, dst, h2t):
    f = pl.kernel(
        _s2_body,
        out_type=jax.ShapeDtypeStruct((2, NTILES, RPT, W2ROW), jnp.float32),
        mesh=_mesh(),
        compiler_params=pltpu.CompilerParams(
            use_tc_tiling_on_sc=False, needs_layout_passes=False),
        scratch_types=[
            pltpu.VMEM((N,), jnp.float32),
            pltpu.VMEM((N,), jnp.float32),
            pltpu.VMEM((CH,), jnp.int32),
            pltpu.VMEM((CH,), jnp.int32),
            pltpu.VMEM((CH,), jnp.int32),
            pltpu.VMEM((CH,), jnp.int32),
            pltpu.VMEM((CH,), jnp.int32),
            pltpu.VMEM((CH,), jnp.int32),
            pltpu.VMEM((CH, C2P), jnp.float32),
            pltpu.VMEM((CH, C2P), jnp.float32),
            pltpu.VMEM((CH, C2P), jnp.float32),
            pltpu.VMEM((CH, W2ROW), jnp.float32),
            pltpu.VMEM((CH, W2ROW), jnp.float32),
            pltpu.VMEM((CH, W2ROW), jnp.float32),
            pltpu.VMEM((CH,), jnp.float32),
            pltpu.VMEM((ZR, W2ROW), jnp.float32),
            pltpu.VMEM_SHARED((N, W2ROW), jnp.float32),
            pltpu.SemaphoreType.DMA,
            pltpu.SemaphoreType.DMA,
            pltpu.SemaphoreType.DMA,
            pltpu.SemaphoreType.DMA,
            pltpu.SemaphoreType.DMA,
            pltpu.SemaphoreType.DMA,
        ],
    )
    return f(asrc2, adst2, src, dst, h2t)


def _k3_body(num_ref, b2_ref, out_ref):
    ssum = num_ref[0] + num_ref[1]
    den = ssum[:, C2P : C2P + 1]
    o = ssum[:, :NCLASS] / (den + 1e-16) + b2_ref[0][None, :]
    m = jnp.max(o, axis=1, keepdims=True)
    zl = o - m
    out_ref[...] = zl - jnp.log(jnp.sum(jnp.exp(zl), axis=1, keepdims=True))


def _k3(num2, b2):
    nb = N // BN
    return pl.pallas_call(
        _k3_body,
        grid=(nb,),
        in_specs=[
            pl.BlockSpec((2, BN, W2ROW), lambda i: (0, i, 0)),
            pl.BlockSpec((1, NCLASS), lambda i: (0, 0)),
        ],
        out_specs=pl.BlockSpec((BN, NCLASS), lambda i: (i, 0)),
        out_shape=jax.ShapeDtypeStruct((N, NCLASS), jnp.float32),
    )(num2, b2)


def kernel(x, edge_index, W1, a_src1, a_dst1, b1, W2, a_src2, a_dst2, b2):
    src = edge_index[0]
    dst = edge_index[1]

    h1t, asrc1, adst1 = _k1(x, W1, a_src1, a_dst1)
    num1 = _s1(asrc1, adst1, src, dst, h1t.reshape(HEADS * N, NHID))
    num1 = num1.reshape(HEADS, N, W1ROW)

    W2p = jnp.pad(W2.reshape(NHID, NCLASS), ((0, 0), (0, C2P - NCLASS)))
    a2sp = jnp.pad(a_src2, ((0, 0), (0, C2P - NCLASS)))
    a2dp = jnp.pad(a_dst2, ((0, 0), (0, C2P - NCLASS)))
    b1r = b1.reshape(1, NHID)
    h2t = _k2a(num1, b1r, W2p)
    asrc2, adst2 = _k2b(h2t, a2sp, a2dp)

    num2 = _s2(asrc2, adst2, src, dst, h2t)
    return _k3(num2.reshape(2, N, W2ROW), b2.reshape(1, NCLASS))


# speedup vs baseline: 1.6846x; 1.6846x over previous
import functools

import jax
import jax.numpy as jnp
from jax import lax
from jax.experimental import pallas as pl
from jax.experimental.pallas import tpu as pltpu
from jax.experimental.pallas import tpu_sc as plsc

N = 10000
E = 320000
NFEAT = 128
NHID = 64
NCLASS = 40
HEADS = 8

C2P = 48
W1ROW = 80
W2ROW = 64

BN = 400
CH = 80
NTILES = 16
EPT = E // NTILES
EPT2 = E // (2 * NTILES)
RPT = N // NTILES
ZR = 25

@functools.lru_cache(maxsize=None)
def _mesh():
    return plsc.VectorSubcoreMesh(
        core_axis_name="c", subcore_axis_name="s", num_cores=2, num_subcores=16)


def _k1_body(x_ref, w1_ref, as_ref, ad_ref, h1_ref, asrc_ref, adst_ref):
    xb = x_ref[...]
    wh = w1_ref[0]
    hh = jnp.dot(xb, wh, preferred_element_type=jnp.float32)
    h1_ref[0] = hh
    asrc_ref[...] = jnp.sum(hh * as_ref[0], axis=1)[None, None, :]
    adst_ref[...] = jnp.sum(hh * ad_ref[0], axis=1)[None, None, :]


def _k1(x, W1, a_src1, a_dst1):
    return pl.pallas_call(
        _k1_body,
        grid=(HEADS,),
        in_specs=[
            pl.BlockSpec((N, NFEAT), lambda h: (0, 0)),
            pl.BlockSpec((1, NFEAT, NHID), lambda h: (h, 0, 0)),
            pl.BlockSpec((1, 1, NHID), lambda h: (h, 0, 0)),
            pl.BlockSpec((1, 1, NHID), lambda h: (h, 0, 0)),
        ],
        out_specs=[
            pl.BlockSpec((1, N, NHID), lambda h: (h, 0, 0)),
            pl.BlockSpec((1, 1, N), lambda h: (h, 0, 0)),
            pl.BlockSpec((1, 1, N), lambda h: (h, 0, 0)),
        ],
        out_shape=[
            jax.ShapeDtypeStruct((HEADS, N, NHID), jnp.float32),
            jax.ShapeDtypeStruct((HEADS, 1, N), jnp.float32),
            jax.ShapeDtypeStruct((HEADS, 1, N), jnp.float32),
        ],
    )(x, W1.transpose(1, 0, 2), a_src1.reshape(HEADS, 1, NHID),
      a_dst1.reshape(HEADS, 1, NHID))


def _zero2d(ref, nrows, ncols):
    z16 = jnp.zeros((16,), jnp.float32)

    def body(r, _):
        for cc in range(ncols // 16):
            ref[r, pl.ds(cc * 16, 16)] = z16
        return _

    lax.fori_loop(0, nrows, body, None)


def _s1_body(asrc_hbm, adst_hbm, src_hbm, dst_hbm, h1t_hbm, out_hbm,
             asrc_v, adst_v, srcb0, srcb1, srcb2, srcb3,
             dstb0, dstb1, dstb2, dstb3, idxb0, idxb1, idxb2, idxb3,
             gb0, gb1, gb2, gb3, stb0, stb1, stb2, stb3,
             ee_v, zbuf_v, num_sh,
             gs0, gs1, gs2, gs3, ss0, ss1, ss2, ss3,
             ls0, ls1, ls2, ls3, ld0, ld1, ld2, ld3):
    c = lax.axis_index("c")
    s = lax.axis_index("s")
    srcb = (srcb0, srcb1, srcb2, srcb3)
    dstb = (dstb0, dstb1, dstb2, dstb3)
    idxb = (idxb0, idxb1, idxb2, idxb3)
    gb = (gb0, gb1, gb2, gb3)
    stb = (stb0, stb1, stb2, stb3)
    gsem = (gs0, gs1, gs2, gs3)
    ssem = (ss0, ss1, ss2, ss3)
    lsem = (ls0, ls1, ls2, ls3)
    ldem = (ld0, ld1, ld2, ld3)
    NCH = EPT // CH
    NG = CH // 16
    iota16 = lax.iota(jnp.int32, 16)
    dencol = jnp.full((16,), NHID, jnp.int32)

    _zero2d(zbuf_v, ZR, W1ROW)
    for b in range(4):
        _zero2d(stb[b], CH, W1ROW)

    def issue_loads(b, k):
        ebase = s * EPT + k * CH
        pltpu.async_copy(src_hbm.at[pl.ds(ebase, CH)], srcb[b], lsem[b])
        pltpu.async_copy(dst_hbm.at[pl.ds(ebase, CH)], dstb[b], ldem[b])

    def wait_loads(b, k):
        ebase = s * EPT + k * CH
        pltpu.make_async_copy(src_hbm.at[pl.ds(ebase, CH)], srcb[b], lsem[b]).wait()
        pltpu.make_async_copy(dst_hbm.at[pl.ds(ebase, CH)], dstb[b], ldem[b]).wait()

    def build_idx_and_gather(b, hbase):
        def bidx(gi, _i):
            idxb[b][pl.ds(gi * 16, 16)] = srcb[b][pl.ds(gi * 16, 16)] + hbase
            return _i

        lax.fori_loop(0, NG, bidx, None)
        pltpu.async_copy(h1t_hbm.at[idxb[b]], gb[b], gsem[b])

    def wait_gather(b):
        pltpu.make_async_copy(h1t_hbm.at[idxb[b]], gb[b], gsem[b]).wait()

    def issue_scatter(b):
        pltpu.async_copy(stb[b], num_sh.at[dstb[b]], ssem[b], add=True)

    def wait_scatter(b):
        pltpu.make_async_copy(stb[b], num_sh.at[dstb[b]], ssem[b]).wait()

    def compute(b):
        def eeg(gi, _i):
            sv = srcb[b][pl.ds(gi * 16, 16)]
            dv = dstb[b][pl.ds(gi * 16, 16)]
            z = plsc.load_gather(asrc_v, [sv]) + plsc.load_gather(adst_v, [dv])
            ee_v[pl.ds(gi * 16, 16)] = jnp.exp(jnp.maximum(z, z * 0.2))
            return _i

        lax.fori_loop(0, NG, eeg, None)

        @plsc.parallel_loop(0, NG, 1)
        def pe(gi):
            r0 = gi * 16
            ee16 = ee_v[pl.ds(r0, 16)]
            plsc.store_scatter(stb[b], [iota16 + r0, dencol], ee16)
            for j in range(16):
                e = r0 + j
                eev = jnp.full((16,), ee16[j], jnp.float32)
                for cc in range(NHID // 16):
                    stb[b][e, pl.ds(cc * 16, 16)] = (
                        gb[b][e, pl.ds(cc * 16, 16)] * eev)

    def head_body(hp, _):
        h = c * (HEADS // 2) + hp
        hbase = h * N

        def zz_body(zz, _z):
            pltpu.sync_copy(zbuf_v, num_sh.at[pl.ds(s * RPT + zz * ZR, ZR)])
            return _z

        lax.fori_loop(0, RPT // ZR, zz_body, None)
        pltpu.sync_copy(asrc_hbm.at[h, 0], asrc_v)
        pltpu.sync_copy(adst_hbm.at[h, 0], adst_v)
        plsc.subcore_barrier()

        issue_loads(0, 0)
        issue_loads(1, 1)
        wait_loads(0, 0)
        build_idx_and_gather(0, hbase)

        def quad(q, _t):
            for b in range(4):
                k = 4 * q + b

                @pl.when(jnp.logical_and(k >= 2, k <= NCH + 1))
                def _w():
                    wait_scatter((b - 2) % 4)

                @pl.when(k + 2 < NCH)
                def _l():
                    issue_loads((b + 2) % 4, k + 2)

                @pl.when(k + 1 < NCH)
                def _g():
                    wait_loads((b + 1) % 4, k + 1)
                    build_idx_and_gather((b + 1) % 4, hbase)

                @pl.when(k < NCH)
                def _c():
                    wait_gather(b)
                    compute(b)
                    issue_scatter(b)
            return _t

        lax.fori_loop(0, (NCH + 5) // 4, quad, None)
        plsc.subcore_barrier()
        pltpu.sync_copy(num_sh.at[pl.ds(s * RPT, RPT)], out_hbm.at[h, s])
        plsc.subcore_barrier()
        return _

    lax.fori_loop(0, HEADS // 2, head_body, None)


def _s1(asrc1, adst1, src, dst, h1t_flat):
    f = pl.kernel(
        _s1_body,
        out_type=jax.ShapeDtypeStruct((HEADS, NTILES, RPT, W1ROW), jnp.float32),
        mesh=_mesh(),
        compiler_params=pltpu.CompilerParams(
            use_tc_tiling_on_sc=False, needs_layout_passes=False),
        scratch_types=(
            [pltpu.VMEM((N,), jnp.float32)] * 2
            + [pltpu.VMEM((CH,), jnp.int32)] * 12
            + [pltpu.VMEM((CH, NHID), jnp.float32)] * 4
            + [pltpu.VMEM((CH, W1ROW), jnp.float32)] * 4
            + [pltpu.VMEM((CH,), jnp.float32)]
            + [pltpu.VMEM((ZR, W1ROW), jnp.float32)]
            + [pltpu.VMEM_SHARED((N, W1ROW), jnp.float32)]
            + [pltpu.SemaphoreType.DMA] * 16
        ),
    )
    return f(asrc1, adst1, src, dst, h1t_flat)


def _k2a_body(num_ref, b1_ref, w2_ref, h2_ref):
    acc = num_ref[...]
    den = acc[:, :, NHID : NHID + 1]
    o = jnp.sum(acc[:, :, :NHID] / (den + 1e-16), axis=0) * (1.0 / HEADS)
    o = o + b1_ref[...]
    h2a = jnp.where(o > 0, o, jnp.exp(o) - 1.0)
    h2_ref[...] = jnp.dot(h2a, w2_ref[...], preferred_element_type=jnp.float32)


def _k2a(num1, b1, W2p):
    nb = N // BN
    return pl.pallas_call(
        _k2a_body,
        grid=(nb,),
        in_specs=[
            pl.BlockSpec((HEADS, BN, W1ROW), lambda i: (0, i, 0)),
            pl.BlockSpec((1, NHID), lambda i: (0, 0)),
            pl.BlockSpec((NHID, C2P), lambda i: (0, 0)),
        ],
        out_specs=pl.BlockSpec((BN, C2P), lambda i: (i, 0)),
        out_shape=jax.ShapeDtypeStruct((N, C2P), jnp.float32),
    )(num1, b1, W2p)


def _k2b_body(h2_ref, a2s_ref, a2d_ref, asrc2_ref, adst2_ref):
    h2 = h2_ref[...]
    asrc2_ref[...] = jnp.sum(h2 * a2s_ref[0][None, :], axis=1)[None, None, :]
    adst2_ref[...] = jnp.sum(h2 * a2d_ref[0][None, :], axis=1)[None, None, :]


def _k2b(h2t, a2sp, a2dp):
    return pl.pallas_call(
        _k2b_body,
        grid=(1,),
        in_specs=[
            pl.BlockSpec((N, C2P), lambda i: (0, 0)),
            pl.BlockSpec((1, C2P), lambda i: (0, 0)),
            pl.BlockSpec((1, C2P), lambda i: (0, 0)),
        ],
        out_specs=[
            pl.BlockSpec((1, 1, N), lambda i: (0, 0, 0)),
            pl.BlockSpec((1, 1, N), lambda i: (0, 0, 0)),
        ],
        out_shape=[
            jax.ShapeDtypeStruct((1, 1, N), jnp.float32),
            jax.ShapeDtypeStruct((1, 1, N), jnp.float32),
        ],
    )(h2t, a2sp, a2dp)


def _s2_body(asrc_hbm, adst_hbm, src_hbm, dst_hbm, h2t_hbm, out_hbm,
             asrc_v, adst_v, srcb0, srcb1, srcb2, srcb3,
             dstb0, dstb1, dstb2, dstb3,
             gb0, gb1, gb2, gb3, stb0, stb1, stb2, stb3,
             ee_v, zbuf_v, num_sh,
             gs0, gs1, gs2, gs3, ss0, ss1, ss2, ss3,
             ls0, ls1, ls2, ls3, ld0, ld1, ld2, ld3):
    c = lax.axis_index("c")
    s = lax.axis_index("s")
    srcb = (srcb0, srcb1, srcb2, srcb3)
    dstb = (dstb0, dstb1, dstb2, dstb3)
    gb = (gb0, gb1, gb2, gb3)
    stb = (stb0, stb1, stb2, stb3)
    gsem = (gs0, gs1, gs2, gs3)
    ssem = (ss0, ss1, ss2, ss3)
    lsem = (ls0, ls1, ls2, ls3)
    ldem = (ld0, ld1, ld2, ld3)
    NCH = EPT2 // CH
    NG = CH // 16
    iota16 = lax.iota(jnp.int32, 16)
    dencol = jnp.full((16,), C2P, jnp.int32)

    _zero2d(zbuf_v, ZR, W2ROW)
    for b in range(4):
        _zero2d(stb[b], CH, W2ROW)

    def zz_body(zz, _z):
        pltpu.sync_copy(zbuf_v, num_sh.at[pl.ds(s * RPT + zz * ZR, ZR)])
        return _z

    lax.fori_loop(0, RPT // ZR, zz_body, None)
    pltpu.sync_copy(asrc_hbm.at[0, 0], asrc_v)
    pltpu.sync_copy(adst_hbm.at[0, 0], adst_v)
    plsc.subcore_barrier()

    def issue_loads(b, k):
        ebase = (c * NTILES + s) * EPT2 + k * CH
        pltpu.async_copy(src_hbm.at[pl.ds(ebase, CH)], srcb[b], lsem[b])
        pltpu.async_copy(dst_hbm.at[pl.ds(ebase, CH)], dstb[b], ldem[b])

    def wait_loads(b, k):
        ebase = (c * NTILES + s) * EPT2 + k * CH
        pltpu.make_async_copy(src_hbm.at[pl.ds(ebase, CH)], srcb[b], lsem[b]).wait()
        pltpu.make_async_copy(dst_hbm.at[pl.ds(ebase, CH)], dstb[b], ldem[b]).wait()

    def issue_gather(b):
        pltpu.async_copy(h2t_hbm.at[srcb[b]], gb[b], gsem[b])

    def wait_gather(b):
        pltpu.make_async_copy(h2t_hbm.at[srcb[b]], gb[b], gsem[b]).wait()

    def issue_scatter(b):
        pltpu.async_copy(stb[b], num_sh.at[dstb[b]], ssem[b], add=True)

    def wait_scatter(b):
        pltpu.make_async_copy(stb[b], num_sh.at[dstb[b]], ssem[b]).wait()

    def compute(b):
        def eeg(gi, _i):
            sv = srcb[b][pl.ds(gi * 16, 16)]
            dv = dstb[b][pl.ds(gi * 16, 16)]
            z = plsc.load_gather(asrc_v, [sv]) + plsc.load_gather(adst_v, [dv])
            ee_v[pl.ds(gi * 16, 16)] = jnp.exp(jnp.maximum(z, z * 0.2))
            return _i

        lax.fori_loop(0, NG, eeg, None)

        @plsc.parallel_loop(0, NG, 1)
        def pe(gi):
            r0 = gi * 16
            ee16 = ee_v[pl.ds(r0, 16)]
            plsc.store_scatter(stb[b], [iota16 + r0, dencol], ee16)
            for j in range(16):
                e = r0 + j
                eev = jnp.full((16,), ee16[j], jnp.float32)
                for cc in range(C2P // 16):
                    stb[b][e, pl.ds(cc * 16, 16)] = (
                        gb[b][e, pl.ds(cc * 16, 16)] * eev)

    issue_loads(0, 0)
    issue_loads(1, 1)
    wait_loads(0, 0)
    issue_gather(0)

    def quad(q, _t):
        for b in range(4):
            k = 4 * q + b

            @pl.when(jnp.logical_and(k >= 2, k <= NCH + 1))
            def _w():
                wait_scatter((b - 2) % 4)

            @pl.when(k + 2 < NCH)
            def _l():
                issue_loads((b + 2) % 4, k + 2)

            @pl.when(k + 1 < NCH)
            def _g():
                wait_loads((b + 1) % 4, k + 1)
                issue_gather((b + 1) % 4)

            @pl.when(k < NCH)
            def _c():
                wait_gather(b)
                compute(b)
                issue_scatter(b)
        return _t

    lax.fori_loop(0, (NCH + 5) // 4, quad, None)
    plsc.subcore_barrier()
    pltpu.sync_copy(num_sh.at[pl.ds(s * RPT, RPT)], out_hbm.at[c, s])


def _s2(asrc2, adst2, src, dst, h2t):
    f = pl.kernel(
        _s2_body,
        out_type=jax.ShapeDtypeStruct((2, NTILES, RPT, W2ROW), jnp.float32),
        mesh=_mesh(),
        compiler_params=pltpu.CompilerParams(
            use_tc_tiling_on_sc=False, needs_layout_passes=False),
        scratch_types=(
            [pltpu.VMEM((N,), jnp.float32)] * 2
            + [pltpu.VMEM((CH,), jnp.int32)] * 8
            + [pltpu.VMEM((CH, C2P), jnp.float32)] * 4
            + [pltpu.VMEM((CH, W2ROW), jnp.float32)] * 4
            + [pltpu.VMEM((CH,), jnp.float32)]
            + [pltpu.VMEM((ZR, W2ROW), jnp.float32)]
            + [pltpu.VMEM_SHARED((N, W2ROW), jnp.float32)]
            + [pltpu.SemaphoreType.DMA] * 16
        ),
    )
    return f(asrc2, adst2, src, dst, h2t)


def _k3_body(num_ref, b2_ref, out_ref):
    ssum = num_ref[0] + num_ref[1]
    den = ssum[:, C2P : C2P + 1]
    o = ssum[:, :NCLASS] / (den + 1e-16) + b2_ref[0][None, :]
    m = jnp.max(o, axis=1, keepdims=True)
    zl = o - m
    out_ref[...] = zl - jnp.log(jnp.sum(jnp.exp(zl), axis=1, keepdims=True))


def _k3(num2, b2):
    nb = N // BN
    return pl.pallas_call(
        _k3_body,
        grid=(nb,),
        in_specs=[
            pl.BlockSpec((2, BN, W2ROW), lambda i: (0, i, 0)),
            pl.BlockSpec((1, NCLASS), lambda i: (0, 0)),
        ],
        out_specs=pl.BlockSpec((BN, NCLASS), lambda i: (i, 0)),
        out_shape=jax.ShapeDtypeStruct((N, NCLASS), jnp.float32),
    )(num2, b2)


def kernel(x, edge_index, W1, a_src1, a_dst1, b1, W2, a_src2, a_dst2, b2):
    src = edge_index[0]
    dst = edge_index[1]

    h1t, asrc1, adst1 = _k1(x, W1, a_src1, a_dst1)
    num1 = _s1(asrc1, adst1, src, dst, h1t.reshape(HEADS * N, NHID))
    num1 = num1.reshape(HEADS, N, W1ROW)

    W2p = jnp.pad(W2.reshape(NHID, NCLASS), ((0, 0), (0, C2P - NCLASS)))
    a2sp = jnp.pad(a_src2, ((0, 0), (0, C2P - NCLASS)))
    a2dp = jnp.pad(a_dst2, ((0, 0), (0, C2P - NCLASS)))
    b1r = b1.reshape(1, NHID)
    h2t = _k2a(num1, b1r, W2p)
    asrc2, adst2 = _k2b(h2t, a2sp, a2dp)

    num2 = _s2(asrc2, adst2, src, dst, h2t)
    return _k3(num2.reshape(2, N, W2ROW), b2.reshape(1, NCLASS))


# speedup vs baseline: 1.7630x; 1.0465x over previous
import functools

import jax
import jax.numpy as jnp
from jax import lax
from jax.experimental import pallas as pl
from jax.experimental.pallas import tpu as pltpu
from jax.experimental.pallas import tpu_sc as plsc

N = 10000
E = 320000
NFEAT = 128
NHID = 64
NCLASS = 40
HEADS = 8

C2P = 48
W1ROW = 80
W2ROW = 64

BN = 400
CH = 80
NTILES = 16
EPT = E // NTILES
EPT2 = E // (2 * NTILES)
RPT = N // NTILES
ZR = 25

@functools.lru_cache(maxsize=None)
def _mesh():
    return plsc.VectorSubcoreMesh(
        core_axis_name="c", subcore_axis_name="s", num_cores=2, num_subcores=16)


def _k1_body(x_ref, w1_ref, as_ref, ad_ref, h1_ref, asrc_ref, adst_ref):
    xb = x_ref[...]
    wh = w1_ref[0]
    hh = jnp.dot(xb, wh, preferred_element_type=jnp.float32)
    h1_ref[0] = hh
    asrc_ref[...] = jnp.sum(hh * as_ref[0], axis=1)[None, None, :]
    adst_ref[...] = jnp.sum(hh * ad_ref[0], axis=1)[None, None, :]


def _k1(x, W1, a_src1, a_dst1):
    return pl.pallas_call(
        _k1_body,
        grid=(HEADS,),
        in_specs=[
            pl.BlockSpec((N, NFEAT), lambda h: (0, 0)),
            pl.BlockSpec((1, NFEAT, NHID), lambda h: (h, 0, 0)),
            pl.BlockSpec((1, 1, NHID), lambda h: (h, 0, 0)),
            pl.BlockSpec((1, 1, NHID), lambda h: (h, 0, 0)),
        ],
        out_specs=[
            pl.BlockSpec((1, N, NHID), lambda h: (h, 0, 0)),
            pl.BlockSpec((1, 1, N), lambda h: (h, 0, 0)),
            pl.BlockSpec((1, 1, N), lambda h: (h, 0, 0)),
        ],
        out_shape=[
            jax.ShapeDtypeStruct((HEADS, N, NHID), jnp.float32),
            jax.ShapeDtypeStruct((HEADS, 1, N), jnp.float32),
            jax.ShapeDtypeStruct((HEADS, 1, N), jnp.float32),
        ],
    )(x, W1.transpose(1, 0, 2), a_src1.reshape(HEADS, 1, NHID),
      a_dst1.reshape(HEADS, 1, NHID))


def _zero2d(ref, nrows, ncols):
    z16 = jnp.zeros((16,), jnp.float32)

    def body(r, _):
        for cc in range(ncols // 16):
            ref[r, pl.ds(cc * 16, 16)] = z16
        return _

    lax.fori_loop(0, nrows, body, None)


def _s1_body(asrc_hbm, adst_hbm, src_hbm, dst_hbm, h1t_hbm, out_hbm,
             asrc_v, adst_v, srcb0, srcb1, srcb2, srcb3,
             dstb0, dstb1, dstb2, dstb3, idxb0, idxb1, idxb2, idxb3,
             dsc0, dsc1, dsc2, dsc3,
             gb0, gb1, gb2, gb3, stb0, stb1, stb2, stb3,
             zbuf_v, num_sh,
             gs0, gs1, gs2, gs3, ss0, ss1, ss2, ss3,
             ls0, ls1, ls2, ls3, ld0, ld1, ld2, ld3):
    c = lax.axis_index("c")
    s = lax.axis_index("s")
    srcb = (srcb0, srcb1, srcb2, srcb3)
    dstb = (dstb0, dstb1, dstb2, dstb3)
    idxb = (idxb0, idxb1, idxb2, idxb3)
    dsc = (dsc0, dsc1, dsc2, dsc3)
    gb = (gb0, gb1, gb2, gb3)
    stb = (stb0, stb1, stb2, stb3)
    gsem = (gs0, gs1, gs2, gs3)
    ssem = (ss0, ss1, ss2, ss3)
    lsem = (ls0, ls1, ls2, ls3)
    ldem = (ld0, ld1, ld2, ld3)
    NCH = EPT // CH
    NG = CH // 16
    iota16 = lax.iota(jnp.int32, 16)
    dencol = jnp.full((16,), NHID, jnp.int32)

    _zero2d(zbuf_v, ZR, W1ROW)
    for b in range(4):
        _zero2d(stb[b], CH, W1ROW)

    def issue_loads(b, k):
        ebase = s * EPT + k * CH
        pltpu.async_copy(src_hbm.at[pl.ds(ebase, CH)], srcb[b], lsem[b])
        pltpu.async_copy(dst_hbm.at[pl.ds(ebase, CH)], dstb[b], ldem[b])

    def wait_loads(b, k):
        ebase = s * EPT + k * CH
        pltpu.make_async_copy(src_hbm.at[pl.ds(ebase, CH)], srcb[b], lsem[b]).wait()
        pltpu.make_async_copy(dst_hbm.at[pl.ds(ebase, CH)], dstb[b], ldem[b]).wait()

    def build_idx_and_gather(b, hbase):
        def bidx(gi, _i):
            idxb[b][pl.ds(gi * 16, 16)] = srcb[b][pl.ds(gi * 16, 16)] + hbase
            return _i

        lax.fori_loop(0, NG, bidx, None)
        pltpu.async_copy(h1t_hbm.at[idxb[b]], gb[b], gsem[b])

    def wait_gather(b):
        pltpu.make_async_copy(h1t_hbm.at[idxb[b]], gb[b], gsem[b]).wait()

    def issue_scatter(b):
        pltpu.async_copy(stb[b], num_sh.at[dsc[b]], ssem[b], add=True)

    def wait_scatter(b):
        pltpu.make_async_copy(stb[b], num_sh.at[dsc[b]], ssem[b]).wait()

    def compute(b):
        @plsc.parallel_loop(0, NG, 1)
        def pe(gi):
            r0 = gi * 16
            sv = srcb[b][pl.ds(r0, 16)]
            dv = dstb[b][pl.ds(r0, 16)]
            dsc[b][pl.ds(r0, 16)] = dv
            z = plsc.load_gather(asrc_v, [sv]) + plsc.load_gather(adst_v, [dv])
            ee16 = jnp.exp(jnp.maximum(z, z * 0.2))
            plsc.store_scatter(stb[b], [iota16 + r0, dencol], ee16)
            for j in range(16):
                e = r0 + j
                eev = jnp.full((16,), ee16[j], jnp.float32)
                for cc in range(NHID // 16):
                    stb[b][e, pl.ds(cc * 16, 16)] = (
                        gb[b][e, pl.ds(cc * 16, 16)] * eev)

    def head_body(hp, _):
        h = c * (HEADS // 2) + hp
        hbase = h * N

        def zz_body(zz, _z):
            pltpu.sync_copy(zbuf_v, num_sh.at[pl.ds(s * RPT + zz * ZR, ZR)])
            return _z

        lax.fori_loop(0, RPT // ZR, zz_body, None)
        pltpu.sync_copy(asrc_hbm.at[h, 0], asrc_v)
        pltpu.sync_copy(adst_hbm.at[h, 0], adst_v)
        plsc.subcore_barrier()

        issue_loads(0, 0)
        issue_loads(1, 1)
        issue_loads(2, 2)
        wait_loads(0, 0)
        build_idx_and_gather(0, hbase)
        wait_loads(1, 1)
        build_idx_and_gather(1, hbase)

        def quad(q, _t):
            for b in range(4):
                k = 4 * q + b

                @pl.when(jnp.logical_and(k >= 2, k <= NCH + 1))
                def _w():
                    wait_scatter((b - 2) % 4)

                @pl.when(k + 3 < NCH)
                def _l():
                    issue_loads((b + 3) % 4, k + 3)

                @pl.when(k + 2 < NCH)
                def _g():
                    wait_loads((b + 2) % 4, k + 2)
                    build_idx_and_gather((b + 2) % 4, hbase)

                @pl.when(k < NCH)
                def _c():
                    wait_gather(b)
                    compute(b)
                    issue_scatter(b)
            return _t

        lax.fori_loop(0, (NCH + 5) // 4, quad, None)
        plsc.subcore_barrier()
        pltpu.sync_copy(num_sh.at[pl.ds(s * RPT, RPT)], out_hbm.at[h, s])
        plsc.subcore_barrier()
        return _

    lax.fori_loop(0, HEADS // 2, head_body, None)


def _s1(asrc1, adst1, src, dst, h1t_flat):
    f = pl.kernel(
        _s1_body,
        out_type=jax.ShapeDtypeStruct((HEADS, NTILES, RPT, W1ROW), jnp.float32),
        mesh=_mesh(),
        compiler_params=pltpu.CompilerParams(
            use_tc_tiling_on_sc=False, needs_layout_passes=False),
        scratch_types=(
            [pltpu.VMEM((N,), jnp.float32)] * 2
            + [pltpu.VMEM((CH,), jnp.int32)] * 16
            + [pltpu.VMEM((CH, NHID), jnp.float32)] * 4
            + [pltpu.VMEM((CH, W1ROW), jnp.float32)] * 4
            + [pltpu.VMEM((ZR, W1ROW), jnp.float32)]
            + [pltpu.VMEM_SHARED((N, W1ROW), jnp.float32)]
            + [pltpu.SemaphoreType.DMA] * 16
        ),
    )
    return f(asrc1, adst1, src, dst, h1t_flat)


def _k2a_body(num_ref, b1_ref, w2_ref, h2_ref):
    acc = num_ref[...]
    den = acc[:, :, NHID : NHID + 1]
    o = jnp.sum(acc[:, :, :NHID] / (den + 1e-16), axis=0) * (1.0 / HEADS)
    o = o + b1_ref[...]
    h2a = jnp.where(o > 0, o, jnp.exp(o) - 1.0)
    h2_ref[...] = jnp.dot(h2a, w2_ref[...], preferred_element_type=jnp.float32)


def _k2a(num1, b1, W2p):
    nb = N // BN
    return pl.pallas_call(
        _k2a_body,
        grid=(nb,),
        in_specs=[
            pl.BlockSpec((HEADS, BN, W1ROW), lambda i: (0, i, 0)),
            pl.BlockSpec((1, NHID), lambda i: (0, 0)),
            pl.BlockSpec((NHID, C2P), lambda i: (0, 0)),
        ],
        out_specs=pl.BlockSpec((BN, C2P), lambda i: (i, 0)),
        out_shape=jax.ShapeDtypeStruct((N, C2P), jnp.float32),
    )(num1, b1, W2p)


def _k2b_body(h2_ref, a2s_ref, a2d_ref, asrc2_ref, adst2_ref):
    h2 = h2_ref[...]
    asrc2_ref[...] = jnp.sum(h2 * a2s_ref[0][None, :], axis=1)[None, None, :]
    adst2_ref[...] = jnp.sum(h2 * a2d_ref[0][None, :], axis=1)[None, None, :]


def _k2b(h2t, a2sp, a2dp):
    return pl.pallas_call(
        _k2b_body,
        grid=(1,),
        in_specs=[
            pl.BlockSpec((N, C2P), lambda i: (0, 0)),
            pl.BlockSpec((1, C2P), lambda i: (0, 0)),
            pl.BlockSpec((1, C2P), lambda i: (0, 0)),
        ],
        out_specs=[
            pl.BlockSpec((1, 1, N), lambda i: (0, 0, 0)),
            pl.BlockSpec((1, 1, N), lambda i: (0, 0, 0)),
        ],
        out_shape=[
            jax.ShapeDtypeStruct((1, 1, N), jnp.float32),
            jax.ShapeDtypeStruct((1, 1, N), jnp.float32),
        ],
    )(h2t, a2sp, a2dp)


def _s2_body(asrc_hbm, adst_hbm, src_hbm, dst_hbm, h2t_hbm, out_hbm,
             asrc_v, adst_v, srcb0, srcb1, srcb2, srcb3,
             dstb0, dstb1, dstb2, dstb3,
             gb0, gb1, gb2, gb3, stb0, stb1, stb2, stb3,
             ee_v, zbuf_v, num_sh,
             gs0, gs1, gs2, gs3, ss0, ss1, ss2, ss3,
             ls0, ls1, ls2, ls3, ld0, ld1, ld2, ld3):
    c = lax.axis_index("c")
    s = lax.axis_index("s")
    srcb = (srcb0, srcb1, srcb2, srcb3)
    dstb = (dstb0, dstb1, dstb2, dstb3)
    gb = (gb0, gb1, gb2, gb3)
    stb = (stb0, stb1, stb2, stb3)
    gsem = (gs0, gs1, gs2, gs3)
    ssem = (ss0, ss1, ss2, ss3)
    lsem = (ls0, ls1, ls2, ls3)
    ldem = (ld0, ld1, ld2, ld3)
    NCH = EPT2 // CH
    NG = CH // 16
    iota16 = lax.iota(jnp.int32, 16)
    dencol = jnp.full((16,), C2P, jnp.int32)

    _zero2d(zbuf_v, ZR, W2ROW)
    for b in range(4):
        _zero2d(stb[b], CH, W2ROW)

    def zz_body(zz, _z):
        pltpu.sync_copy(zbuf_v, num_sh.at[pl.ds(s * RPT + zz * ZR, ZR)])
        return _z

    lax.fori_loop(0, RPT // ZR, zz_body, None)
    pltpu.sync_copy(asrc_hbm.at[0, 0], asrc_v)
    pltpu.sync_copy(adst_hbm.at[0, 0], adst_v)
    plsc.subcore_barrier()

    def issue_loads(b, k):
        ebase = (c * NTILES + s) * EPT2 + k * CH
        pltpu.async_copy(src_hbm.at[pl.ds(ebase, CH)], srcb[b], lsem[b])
        pltpu.async_copy(dst_hbm.at[pl.ds(ebase, CH)], dstb[b], ldem[b])

    def wait_loads(b, k):
        ebase = (c * NTILES + s) * EPT2 + k * CH
        pltpu.make_async_copy(src_hbm.at[pl.ds(ebase, CH)], srcb[b], lsem[b]).wait()
        pltpu.make_async_copy(dst_hbm.at[pl.ds(ebase, CH)], dstb[b], ldem[b]).wait()

    def issue_gather(b):
        pltpu.async_copy(h2t_hbm.at[srcb[b]], gb[b], gsem[b])

    def wait_gather(b):
        pltpu.make_async_copy(h2t_hbm.at[srcb[b]], gb[b], gsem[b]).wait()

    def issue_scatter(b):
        pltpu.async_copy(stb[b], num_sh.at[dstb[b]], ssem[b], add=True)

    def wait_scatter(b):
        pltpu.make_async_copy(stb[b], num_sh.at[dstb[b]], ssem[b]).wait()

    def compute(b):
        def eeg(gi, _i):
            sv = srcb[b][pl.ds(gi * 16, 16)]
            dv = dstb[b][pl.ds(gi * 16, 16)]
            z = plsc.load_gather(asrc_v, [sv]) + plsc.load_gather(adst_v, [dv])
            ee_v[pl.ds(gi * 16, 16)] = jnp.exp(jnp.maximum(z, z * 0.2))
            return _i

        lax.fori_loop(0, NG, eeg, None)

        @plsc.parallel_loop(0, NG, 1)
        def pe(gi):
            r0 = gi * 16
            ee16 = ee_v[pl.ds(r0, 16)]
            plsc.store_scatter(stb[b], [iota16 + r0, dencol], ee16)
            for j in range(16):
                e = r0 + j
                eev = jnp.full((16,), ee16[j], jnp.float32)
                for cc in range(C2P // 16):
                    stb[b][e, pl.ds(cc * 16, 16)] = (
                        gb[b][e, pl.ds(cc * 16, 16)] * eev)

    issue_loads(0, 0)
    issue_loads(1, 1)
    wait_loads(0, 0)
    issue_gather(0)

    def quad(q, _t):
        for b in range(4):
            k = 4 * q + b

            @pl.when(jnp.logical_and(k >= 2, k <= NCH + 1))
            def _w():
                wait_scatter((b - 2) % 4)

            @pl.when(k + 2 < NCH)
            def _l():
                issue_loads((b + 2) % 4, k + 2)

            @pl.when(k + 1 < NCH)
            def _g():
                wait_loads((b + 1) % 4, k + 1)
                issue_gather((b + 1) % 4)

            @pl.when(k < NCH)
            def _c():
                wait_gather(b)
                compute(b)
                issue_scatter(b)
        return _t

    lax.fori_loop(0, (NCH + 5) // 4, quad, None)
    plsc.subcore_barrier()
    pltpu.sync_copy(num_sh.at[pl.ds(s * RPT, RPT)], out_hbm.at[c, s])


def _s2(asrc2, adst2, src, dst, h2t):
    f = pl.kernel(
        _s2_body,
        out_type=jax.ShapeDtypeStruct((2, NTILES, RPT, W2ROW), jnp.float32),
        mesh=_mesh(),
        compiler_params=pltpu.CompilerParams(
            use_tc_tiling_on_sc=False, needs_layout_passes=False),
        scratch_types=(
            [pltpu.VMEM((N,), jnp.float32)] * 2
            + [pltpu.VMEM((CH,), jnp.int32)] * 8
            + [pltpu.VMEM((CH, C2P), jnp.float32)] * 4
            + [pltpu.VMEM((CH, W2ROW), jnp.float32)] * 4
            + [pltpu.VMEM((CH,), jnp.float32)]
            + [pltpu.VMEM((ZR, W2ROW), jnp.float32)]
            + [pltpu.VMEM_SHARED((N, W2ROW), jnp.float32)]
            + [pltpu.SemaphoreType.DMA] * 16
        ),
    )
    return f(asrc2, adst2, src, dst, h2t)


def _k3_body(num_ref, b2_ref, out_ref):
    ssum = num_ref[0] + num_ref[1]
    den = ssum[:, C2P : C2P + 1]
    o = ssum[:, :NCLASS] / (den + 1e-16) + b2_ref[0][None, :]
    m = jnp.max(o, axis=1, keepdims=True)
    zl = o - m
    out_ref[...] = zl - jnp.log(jnp.sum(jnp.exp(zl), axis=1, keepdims=True))


def _k3(num2, b2):
    nb = N // BN
    return pl.pallas_call(
        _k3_body,
        grid=(nb,),
        in_specs=[
            pl.BlockSpec((2, BN, W2ROW), lambda i: (0, i, 0)),
            pl.BlockSpec((1, NCLASS), lambda i: (0, 0)),
        ],
        out_specs=pl.BlockSpec((BN, NCLASS), lambda i: (i, 0)),
        out_shape=jax.ShapeDtypeStruct((N, NCLASS), jnp.float32),
    )(num2, b2)


def kernel(x, edge_index, W1, a_src1, a_dst1, b1, W2, a_src2, a_dst2, b2):
    src = edge_index[0]
    dst = edge_index[1]

    h1t, asrc1, adst1 = _k1(x, W1, a_src1, a_dst1)
    num1 = _s1(asrc1, adst1, src, dst, h1t.reshape(HEADS * N, NHID))
    num1 = num1.reshape(HEADS, N, W1ROW)

    W2p = jnp.pad(W2.reshape(NHID, NCLASS), ((0, 0), (0, C2P - NCLASS)))
    a2sp = jnp.pad(a_src2, ((0, 0), (0, C2P - NCLASS)))
    a2dp = jnp.pad(a_dst2, ((0, 0), (0, C2P - NCLASS)))
    b1r = b1.reshape(1, NHID)
    h2t = _k2a(num1, b1r, W2p)
    asrc2, adst2 = _k2b(h2t, a2sp, a2dp)

    num2 = _s2(asrc2, adst2, src, dst, h2t)
    return _k3(num2.reshape(2, N, W2ROW), b2.reshape(1, NCLASS))


# speedup vs baseline: 1.9101x; 1.0834x over previous
import functools

import jax
import jax.numpy as jnp
from jax import lax
from jax.experimental import pallas as pl
from jax.experimental.pallas import tpu as pltpu
from jax.experimental.pallas import tpu_sc as plsc

N = 10000
E = 320000
NFEAT = 128
NHID = 64
NCLASS = 40
HEADS = 8

C2P = 48
W1ROW = 80
W2ROW = 64

BN = 400
CH = 80
NTILES = 16
EPT = E // NTILES
EPT2 = E // (2 * NTILES)
RPT = N // NTILES
ZR = 25

@functools.lru_cache(maxsize=None)
def _mesh():
    return plsc.VectorSubcoreMesh(
        core_axis_name="c", subcore_axis_name="s", num_cores=2, num_subcores=16)


def _k1_body(x_ref, w1_ref, as_ref, ad_ref, h1_ref, asrc_ref, adst_ref):
    xb = x_ref[...]
    wh = w1_ref[0]
    hh = jnp.dot(xb, wh, preferred_element_type=jnp.float32)
    h1_ref[0] = hh
    asrc_ref[...] = jnp.sum(hh * as_ref[0], axis=1)[None, None, :]
    adst_ref[...] = jnp.sum(hh * ad_ref[0], axis=1)[None, None, :]


def _k1(x, W1, a_src1, a_dst1):
    return pl.pallas_call(
        _k1_body,
        grid=(HEADS,),
        in_specs=[
            pl.BlockSpec((N, NFEAT), lambda h: (0, 0)),
            pl.BlockSpec((1, NFEAT, NHID), lambda h: (h, 0, 0)),
            pl.BlockSpec((1, 1, NHID), lambda h: (h, 0, 0)),
            pl.BlockSpec((1, 1, NHID), lambda h: (h, 0, 0)),
        ],
        out_specs=[
            pl.BlockSpec((1, N, NHID), lambda h: (h, 0, 0)),
            pl.BlockSpec((1, 1, N), lambda h: (h, 0, 0)),
            pl.BlockSpec((1, 1, N), lambda h: (h, 0, 0)),
        ],
        out_shape=[
            jax.ShapeDtypeStruct((HEADS, N, NHID), jnp.float32),
            jax.ShapeDtypeStruct((HEADS, 1, N), jnp.float32),
            jax.ShapeDtypeStruct((HEADS, 1, N), jnp.float32),
        ],
    )(x, W1.transpose(1, 0, 2), a_src1.reshape(HEADS, 1, NHID),
      a_dst1.reshape(HEADS, 1, NHID))


def _zero2d(ref, nrows, ncols):
    z16 = jnp.zeros((16,), jnp.float32)

    def body(r, _):
        for cc in range(ncols // 16):
            ref[r, pl.ds(cc * 16, 16)] = z16
        return _

    lax.fori_loop(0, nrows, body, None)


def _s1_body(asrc_hbm, adst_hbm, src_hbm, dst_hbm, h1t_hbm, out_hbm,
             asrc_v, adst_v, srcb0, srcb1, srcb2, srcb3,
             dstb0, dstb1, dstb2, dstb3, idxb0, idxb1, idxb2, idxb3,
             dsc0, dsc1, dsc2, dsc3,
             gb0, gb1, gb2, gb3, stb0, stb1, stb2, stb3,
             zbuf_v, num_sh,
             gs0, gs1, gs2, gs3, ss0, ss1, ss2, ss3,
             ls0, ls1, ls2, ls3, ld0, ld1, ld2, ld3):
    c = lax.axis_index("c")
    s = lax.axis_index("s")
    srcb = (srcb0, srcb1, srcb2, srcb3)
    dstb = (dstb0, dstb1, dstb2, dstb3)
    idxb = (idxb0, idxb1, idxb2, idxb3)
    dsc = (dsc0, dsc1, dsc2, dsc3)
    gb = (gb0, gb1, gb2, gb3)
    stb = (stb0, stb1, stb2, stb3)
    gsem = (gs0, gs1, gs2, gs3)
    ssem = (ss0, ss1, ss2, ss3)
    lsem = (ls0, ls1, ls2, ls3)
    ldem = (ld0, ld1, ld2, ld3)
    NCH = EPT // CH
    NG = CH // 16
    iota16 = lax.iota(jnp.int32, 16)
    dencol = jnp.full((16,), NHID, jnp.int32)

    _zero2d(zbuf_v, ZR, W1ROW)
    for b in range(4):
        _zero2d(stb[b], CH, W1ROW)

    def issue_loads(b, k):
        ebase = s * EPT + k * CH
        pltpu.async_copy(src_hbm.at[pl.ds(ebase, CH)], srcb[b], lsem[b])
        pltpu.async_copy(dst_hbm.at[pl.ds(ebase, CH)], dstb[b], ldem[b])

    def wait_loads(b, k):
        ebase = s * EPT + k * CH
        pltpu.make_async_copy(src_hbm.at[pl.ds(ebase, CH)], srcb[b], lsem[b]).wait()
        pltpu.make_async_copy(dst_hbm.at[pl.ds(ebase, CH)], dstb[b], ldem[b]).wait()

    def build_idx_and_gather(b, hbase):
        def bidx(gi, _i):
            idxb[b][pl.ds(gi * 16, 16)] = srcb[b][pl.ds(gi * 16, 16)] + hbase
            return _i

        lax.fori_loop(0, NG, bidx, None)
        pltpu.async_copy(h1t_hbm.at[idxb[b]], gb[b], gsem[b])

    def wait_gather(b):
        pltpu.make_async_copy(h1t_hbm.at[idxb[b]], gb[b], gsem[b]).wait()

    def issue_scatter(b):
        pltpu.async_copy(stb[b], num_sh.at[dsc[b]], ssem[b], add=True)

    def wait_scatter(b):
        pltpu.make_async_copy(stb[b], num_sh.at[dsc[b]], ssem[b]).wait()

    def compute(b):
        @plsc.parallel_loop(0, NG, 1, unroll=2)
        def pe(gi):
            r0 = gi * 16
            sv = srcb[b][pl.ds(r0, 16)]
            dv = dstb[b][pl.ds(r0, 16)]
            dsc[b][pl.ds(r0, 16)] = dv
            z = plsc.load_gather(asrc_v, [sv]) + plsc.load_gather(adst_v, [dv])
            ee16 = jnp.exp(jnp.maximum(z, z * 0.2))
            plsc.store_scatter(stb[b], [iota16 + r0, dencol], ee16)
            for j in range(16):
                e = r0 + j
                eev = jnp.full((16,), ee16[j], jnp.float32)
                for cc in range(NHID // 16):
                    stb[b][e, pl.ds(cc * 16, 16)] = (
                        gb[b][e, pl.ds(cc * 16, 16)] * eev)

    def head_body(hp, _):
        h = c * (HEADS // 2) + hp
        hbase = h * N

        def zz_body(zz, _z):
            pltpu.sync_copy(zbuf_v, num_sh.at[pl.ds(s * RPT + zz * ZR, ZR)])
            return _z

        lax.fori_loop(0, RPT // ZR, zz_body, None)
        pltpu.sync_copy(asrc_hbm.at[h, 0], asrc_v)
        pltpu.sync_copy(adst_hbm.at[h, 0], adst_v)
        plsc.subcore_barrier()

        issue_loads(0, 0)
        issue_loads(1, 1)
        issue_loads(2, 2)
        wait_loads(0, 0)
        build_idx_and_gather(0, hbase)
        wait_loads(1, 1)
        build_idx_and_gather(1, hbase)

        def quad(q, _t):
            for b in range(4):
                k = 4 * q + b

                @pl.when(jnp.logical_and(k >= 2, k <= NCH + 1))
                def _w():
                    wait_scatter((b - 2) % 4)

                @pl.when(k + 3 < NCH)
                def _l():
                    issue_loads((b + 3) % 4, k + 3)

                @pl.when(k + 2 < NCH)
                def _g():
                    wait_loads((b + 2) % 4, k + 2)
                    build_idx_and_gather((b + 2) % 4, hbase)

                @pl.when(k < NCH)
                def _c():
                    wait_gather(b)
                    compute(b)
                    issue_scatter(b)
            return _t

        lax.fori_loop(0, (NCH + 5) // 4, quad, None)
        plsc.subcore_barrier()
        pltpu.sync_copy(num_sh.at[pl.ds(s * RPT, RPT)], out_hbm.at[h, s])
        plsc.subcore_barrier()
        return _

    lax.fori_loop(0, HEADS // 2, head_body, None)


def _s1(asrc1, adst1, src, dst, h1t_flat):
    f = pl.kernel(
        _s1_body,
        out_type=jax.ShapeDtypeStruct((HEADS, NTILES, RPT, W1ROW), jnp.float32),
        mesh=_mesh(),
        compiler_params=pltpu.CompilerParams(
            use_tc_tiling_on_sc=False, needs_layout_passes=False),
        scratch_types=(
            [pltpu.VMEM((N,), jnp.float32)] * 2
            + [pltpu.VMEM((CH,), jnp.int32)] * 16
            + [pltpu.VMEM((CH, NHID), jnp.float32)] * 4
            + [pltpu.VMEM((CH, W1ROW), jnp.float32)] * 4
            + [pltpu.VMEM((ZR, W1ROW), jnp.float32)]
            + [pltpu.VMEM_SHARED((N, W1ROW), jnp.float32)]
            + [pltpu.SemaphoreType.DMA] * 16
        ),
    )
    return f(asrc1, adst1, src, dst, h1t_flat)


def _k2a_body(num_ref, b1_ref, w2_ref, h2_ref):
    acc = num_ref[...]
    den = acc[:, :, NHID : NHID + 1]
    o = jnp.sum(acc[:, :, :NHID] / (den + 1e-16), axis=0) * (1.0 / HEADS)
    o = o + b1_ref[...]
    h2a = jnp.where(o > 0, o, jnp.exp(o) - 1.0)
    h2_ref[...] = jnp.dot(h2a, w2_ref[...], preferred_element_type=jnp.float32)


def _k2a(num1, b1, W2p):
    nb = N // BN
    return pl.pallas_call(
        _k2a_body,
        grid=(nb,),
        in_specs=[
            pl.BlockSpec((HEADS, BN, W1ROW), lambda i: (0, i, 0)),
            pl.BlockSpec((1, NHID), lambda i: (0, 0)),
            pl.BlockSpec((NHID, C2P), lambda i: (0, 0)),
        ],
        out_specs=pl.BlockSpec((BN, C2P), lambda i: (i, 0)),
        out_shape=jax.ShapeDtypeStruct((N, C2P), jnp.float32),
    )(num1, b1, W2p)


def _k2b_body(h2_ref, a2s_ref, a2d_ref, asrc2_ref, adst2_ref):
    h2 = h2_ref[...]
    asrc2_ref[...] = jnp.sum(h2 * a2s_ref[0][None, :], axis=1)[None, None, :]
    adst2_ref[...] = jnp.sum(h2 * a2d_ref[0][None, :], axis=1)[None, None, :]


def _k2b(h2t, a2sp, a2dp):
    return pl.pallas_call(
        _k2b_body,
        grid=(1,),
        in_specs=[
            pl.BlockSpec((N, C2P), lambda i: (0, 0)),
            pl.BlockSpec((1, C2P), lambda i: (0, 0)),
            pl.BlockSpec((1, C2P), lambda i: (0, 0)),
        ],
        out_specs=[
            pl.BlockSpec((1, 1, N), lambda i: (0, 0, 0)),
            pl.BlockSpec((1, 1, N), lambda i: (0, 0, 0)),
        ],
        out_shape=[
            jax.ShapeDtypeStruct((1, 1, N), jnp.float32),
            jax.ShapeDtypeStruct((1, 1, N), jnp.float32),
        ],
    )(h2t, a2sp, a2dp)


def _s2_body(asrc_hbm, adst_hbm, src_hbm, dst_hbm, h2t_hbm, out_hbm,
             asrc_v, adst_v, srcb0, srcb1, srcb2, srcb3,
             dstb0, dstb1, dstb2, dstb3,
             gb0, gb1, gb2, gb3, stb0, stb1, stb2, stb3,
             ee_v, zbuf_v, num_sh,
             gs0, gs1, gs2, gs3, ss0, ss1, ss2, ss3,
             ls0, ls1, ls2, ls3, ld0, ld1, ld2, ld3):
    c = lax.axis_index("c")
    s = lax.axis_index("s")
    srcb = (srcb0, srcb1, srcb2, srcb3)
    dstb = (dstb0, dstb1, dstb2, dstb3)
    gb = (gb0, gb1, gb2, gb3)
    stb = (stb0, stb1, stb2, stb3)
    gsem = (gs0, gs1, gs2, gs3)
    ssem = (ss0, ss1, ss2, ss3)
    lsem = (ls0, ls1, ls2, ls3)
    ldem = (ld0, ld1, ld2, ld3)
    NCH = EPT2 // CH
    NG = CH // 16
    iota16 = lax.iota(jnp.int32, 16)
    dencol = jnp.full((16,), C2P, jnp.int32)

    _zero2d(zbuf_v, ZR, W2ROW)
    for b in range(4):
        _zero2d(stb[b], CH, W2ROW)

    def zz_body(zz, _z):
        pltpu.sync_copy(zbuf_v, num_sh.at[pl.ds(s * RPT + zz * ZR, ZR)])
        return _z

    lax.fori_loop(0, RPT // ZR, zz_body, None)
    pltpu.sync_copy(asrc_hbm.at[0, 0], asrc_v)
    pltpu.sync_copy(adst_hbm.at[0, 0], adst_v)
    plsc.subcore_barrier()

    def issue_loads(b, k):
        ebase = (c * NTILES + s) * EPT2 + k * CH
        pltpu.async_copy(src_hbm.at[pl.ds(ebase, CH)], srcb[b], lsem[b])
        pltpu.async_copy(dst_hbm.at[pl.ds(ebase, CH)], dstb[b], ldem[b])

    def wait_loads(b, k):
        ebase = (c * NTILES + s) * EPT2 + k * CH
        pltpu.make_async_copy(src_hbm.at[pl.ds(ebase, CH)], srcb[b], lsem[b]).wait()
        pltpu.make_async_copy(dst_hbm.at[pl.ds(ebase, CH)], dstb[b], ldem[b]).wait()

    def issue_gather(b):
        pltpu.async_copy(h2t_hbm.at[srcb[b]], gb[b], gsem[b])

    def wait_gather(b):
        pltpu.make_async_copy(h2t_hbm.at[srcb[b]], gb[b], gsem[b]).wait()

    def issue_scatter(b):
        pltpu.async_copy(stb[b], num_sh.at[dstb[b]], ssem[b], add=True)

    def wait_scatter(b):
        pltpu.make_async_copy(stb[b], num_sh.at[dstb[b]], ssem[b]).wait()

    def compute(b):
        def eeg(gi, _i):
            sv = srcb[b][pl.ds(gi * 16, 16)]
            dv = dstb[b][pl.ds(gi * 16, 16)]
            z = plsc.load_gather(asrc_v, [sv]) + plsc.load_gather(adst_v, [dv])
            ee_v[pl.ds(gi * 16, 16)] = jnp.exp(jnp.maximum(z, z * 0.2))
            return _i

        lax.fori_loop(0, NG, eeg, None)

        @plsc.parallel_loop(0, NG, 1)
        def pe(gi):
            r0 = gi * 16
            ee16 = ee_v[pl.ds(r0, 16)]
            plsc.store_scatter(stb[b], [iota16 + r0, dencol], ee16)
            for j in range(16):
                e = r0 + j
                eev = jnp.full((16,), ee16[j], jnp.float32)
                for cc in range(C2P // 16):
                    stb[b][e, pl.ds(cc * 16, 16)] = (
                        gb[b][e, pl.ds(cc * 16, 16)] * eev)

    issue_loads(0, 0)
    issue_loads(1, 1)
    wait_loads(0, 0)
    issue_gather(0)

    def quad(q, _t):
        for b in range(4):
            k = 4 * q + b

            @pl.when(jnp.logical_and(k >= 2, k <= NCH + 1))
            def _w():
                wait_scatter((b - 2) % 4)

            @pl.when(k + 2 < NCH)
            def _l():
                issue_loads((b + 2) % 4, k + 2)

            @pl.when(k + 1 < NCH)
            def _g():
                wait_loads((b + 1) % 4, k + 1)
                issue_gather((b + 1) % 4)

            @pl.when(k < NCH)
            def _c():
                wait_gather(b)
                compute(b)
                issue_scatter(b)
        return _t

    lax.fori_loop(0, (NCH + 5) // 4, quad, None)
    plsc.subcore_barrier()
    pltpu.sync_copy(num_sh.at[pl.ds(s * RPT, RPT)], out_hbm.at[c, s])


def _s2(asrc2, adst2, src, dst, h2t):
    f = pl.kernel(
        _s2_body,
        out_type=jax.ShapeDtypeStruct((2, NTILES, RPT, W2ROW), jnp.float32),
        mesh=_mesh(),
        compiler_params=pltpu.CompilerParams(
            use_tc_tiling_on_sc=False, needs_layout_passes=False),
        scratch_types=(
            [pltpu.VMEM((N,), jnp.float32)] * 2
            + [pltpu.VMEM((CH,), jnp.int32)] * 8
            + [pltpu.VMEM((CH, C2P), jnp.float32)] * 4
            + [pltpu.VMEM((CH, W2ROW), jnp.float32)] * 4
            + [pltpu.VMEM((CH,), jnp.float32)]
            + [pltpu.VMEM((ZR, W2ROW), jnp.float32)]
            + [pltpu.VMEM_SHARED((N, W2ROW), jnp.float32)]
            + [pltpu.SemaphoreType.DMA] * 16
        ),
    )
    return f(asrc2, adst2, src, dst, h2t)


def _k3_body(num_ref, b2_ref, out_ref):
    ssum = num_ref[0] + num_ref[1]
    den = ssum[:, C2P : C2P + 1]
    o = ssum[:, :NCLASS] / (den + 1e-16) + b2_ref[0][None, :]
    m = jnp.max(o, axis=1, keepdims=True)
    zl = o - m
    out_ref[...] = zl - jnp.log(jnp.sum(jnp.exp(zl), axis=1, keepdims=True))


def _k3(num2, b2):
    nb = N // BN
    return pl.pallas_call(
        _k3_body,
        grid=(nb,),
        in_specs=[
            pl.BlockSpec((2, BN, W2ROW), lambda i: (0, i, 0)),
            pl.BlockSpec((1, NCLASS), lambda i: (0, 0)),
        ],
        out_specs=pl.BlockSpec((BN, NCLASS), lambda i: (i, 0)),
        out_shape=jax.ShapeDtypeStruct((N, NCLASS), jnp.float32),
    )(num2, b2)


def kernel(x, edge_index, W1, a_src1, a_dst1, b1, W2, a_src2, a_dst2, b2):
    src = edge_index[0]
    dst = edge_index[1]

    h1t, asrc1, adst1 = _k1(x, W1, a_src1, a_dst1)
    num1 = _s1(asrc1, adst1, src, dst, h1t.reshape(HEADS * N, NHID))
    num1 = num1.reshape(HEADS, N, W1ROW)

    W2p = jnp.pad(W2.reshape(NHID, NCLASS), ((0, 0), (0, C2P - NCLASS)))
    a2sp = jnp.pad(a_src2, ((0, 0), (0, C2P - NCLASS)))
    a2dp = jnp.pad(a_dst2, ((0, 0), (0, C2P - NCLASS)))
    b1r = b1.reshape(1, NHID)
    h2t = _k2a(num1, b1r, W2p)
    asrc2, adst2 = _k2b(h2t, a2sp, a2dp)

    num2 = _s2(asrc2, adst2, src, dst, h2t)
    return _k3(num2.reshape(2, N, W2ROW), b2.reshape(1, NCLASS))
